# baseline (speedup 1.0000x reference)
import functools

import jax
import jax.numpy as jnp
from jax import lax
from jax.experimental import pallas as pl
from jax.experimental.pallas import tpu as pltpu
from jax.experimental.pallas import tpu_sc as plsc

VOCAB = 1000000
DIM = 64
B = 16384
NEG = 20
J = NEG + 1
NC = 2
NS = 16
NW = NC * NS
BPW = B // NW
QT = 4
QCH = BPW // QT
LANES = 16
W2 = 2 * DIM


@functools.partial(
    pl.kernel,
    mesh=plsc.VectorSubcoreMesh(core_axis_name="c", subcore_axis_name="s"),
    compiler_params=pltpu.CompilerParams(needs_layout_passes=False),
    out_type=jax.ShapeDtypeStruct((NW, J * BPW), jnp.float32),
    scratch_types=[
        pltpu.VMEM((QT, QCH), jnp.int32),
        pltpu.VMEM((QT, QCH), jnp.int32),
        pltpu.VMEM((J, QT, QCH), jnp.int32),
        pltpu.VMEM((J, QT, QCH), jnp.int32),
        pltpu.VMEM((QCH, W2), jnp.float32),
        pltpu.VMEM((2, QCH, W2), jnp.float32),
        pltpu.VMEM((J * BPW,), jnp.float32),
        pltpu.SemaphoreType.DMA,
        pltpu.SemaphoreType.DMA,
        pltpu.SemaphoreType.DMA,
    ],
)
def _sc_scores(tih_hbm, tio_hbm, cih_hbm, cio_hbm, wt_hbm, wc_hbm, out_hbm,
               tih_v, tio_v, cih_v, cio_v, t_rows, r_buf, scores_v,
               sem0, sem1, semt):
    wid = lax.axis_index("s") * NC + lax.axis_index("c")

    pltpu.sync_copy(tih_hbm.at[wid], tih_v)
    pltpu.sync_copy(tio_hbm.at[wid], tio_v)
    pltpu.sync_copy(cih_hbm.at[:, wid], cih_v)
    pltpu.sync_copy(cio_hbm.at[:, wid], cio_v)

    lane = jnp.arange(LANES, dtype=jnp.int32)
    sems = (sem0, sem1)

    for qt in range(QT):
        pltpu.async_copy(wt_hbm.at[tih_v.at[qt]], t_rows, semt).wait()
        pltpu.async_copy(wc_hbm.at[cih_v.at[0, qt]], r_buf.at[0], sems[0])

        def compute(j, b, qt=qt):
            def blk_body(bb, c):
                rows = bb * LANES + lane
                tpar = (tio_v[qt, pl.ds(bb * LANES, LANES)] & 1) * DIM
                rpar = (cio_v[j, qt, pl.ds(bb * LANES, LANES)] & 1) * DIM
                acc = jnp.zeros((LANES,), jnp.float32)
                for d in range(DIM):
                    acc = acc + (plsc.load_gather(t_rows, [rows, tpar + d])
                                 * plsc.load_gather(r_buf.at[b],
                                                    [rows, rpar + d]))
                scores_v[pl.ds(j * BPW + qt * QCH + bb * LANES, LANES)] = acc
                return c
            lax.fori_loop(0, QCH // LANES, blk_body, 0)

        def j_body(p, carry, qt=qt):
            for b in range(2):
                j = p * 2 + b

                @pl.when(j < J)
                def _():
                    pltpu.make_async_copy(wc_hbm.at[pl.ds(0, QCH)],
                                          r_buf.at[b], sems[b]).wait()

                    @pl.when(j + 1 < J)
                    def _():
                        pltpu.async_copy(wc_hbm.at[cih_v.at[j + 1, qt]],
                                         r_buf.at[1 - b], sems[1 - b])

                    compute(j, b)
            return carry

        lax.fori_loop(0, (J + 1) // 2, j_body, 0)

    pltpu.sync_copy(scores_v, out_hbm.at[wid])


def _tc_loss_body(s_ref, o_ref):
    s = s_ref[...]
    row = lax.broadcasted_iota(jnp.int32, s.shape, 0)
    x = jnp.where(row % J == 0, -s, s)
    sp = jnp.maximum(x, 0.0) + jnp.log1p(jnp.exp(-jnp.abs(x)))
    o_ref[0, 0] = jnp.sum(sp) * (1.0 / B)


def kernel(target, context, negatives, W_target, W_context):
    tgt = target.astype(jnp.int32)
    cn = jnp.concatenate(
        [context.astype(jnp.int32)[None, :], negatives.astype(jnp.int32).T],
        axis=0)
    tih = (tgt >> 1).reshape(NW, QT, QCH)
    tio = tgt.reshape(NW, QT, QCH)
    cih = (cn >> 1).reshape(J, NW, QT, QCH)
    cio = cn.reshape(J, NW, QT, QCH)
    wt2 = W_target.reshape(VOCAB // 2, W2)
    wc2 = W_context.reshape(VOCAB // 2, W2)

    scores = _sc_scores(tih, tio, cih, cio, wt2, wc2)

    loss = pl.pallas_call(
        _tc_loss_body,
        out_shape=jax.ShapeDtypeStruct((1, 1), jnp.float32),
        out_specs=pl.BlockSpec(memory_space=pltpu.SMEM),
    )(scores.reshape(NW * J, BPW))
    return loss[0, 0]
